# baseline (speedup 1.0000x reference)
import functools

import jax
import jax.numpy as jnp
from jax import lax
from jax.experimental import pallas as pl
from jax.experimental.pallas import tpu as pltpu, tpu_sc as plsc

NC = 2
NS = 16
NW = NC * NS

CHUNK = 128


def _make_sc_kernel(n_rows, chars, vocab, dim):
    rows_per_w = n_rows // NW
    n_chunks = rows_per_w // CHUNK
    assert n_chunks % 2 == 0
    idx_rows = (CHUNK * chars) // 128
    w_idx_rows = n_chunks * idx_rows

    mesh = plsc.VectorSubcoreMesh(core_axis_name="c", subcore_axis_name="s")

    @functools.partial(
        pl.kernel,
        mesh=mesh,
        compiler_params=pltpu.CompilerParams(
            use_tc_tiling_on_sc=False, needs_layout_passes=False),
        out_type=jax.ShapeDtypeStruct((n_rows, dim), jnp.float32),
        scratch_types=[
            pltpu.VMEM((w_idx_rows * 128,), jnp.int32),
            pltpu.VMEM((2 * idx_rows * 128,), jnp.int32),
            pltpu.VMEM((2, CHUNK, dim), jnp.float32),
            pltpu.SemaphoreType.DMA,
            pltpu.SemaphoreType.DMA,
            pltpu.SemaphoreType.DMA,
            pltpu.SemaphoreType.DMA,
            pltpu.SemaphoreType.DMA,
        ],
    )
    def embed_sum(idx_hbm, table_hbm, out_hbm, idx_raw, idx_t, acc_v,
                  sem_i, sem_g0, sem_g1, sem_o0, sem_o1):
        wid = lax.axis_index("s") * NC + lax.axis_index("c")
        sem_g = [sem_g0, sem_g1]
        sem_o = [sem_o0, sem_o1]

        i0 = pl.multiple_of(wid * w_idx_rows * 128, 8)
        pltpu.sync_copy(idx_hbm.at[pl.ds(i0, w_idx_rows * 128)], idx_raw)

        lanes = lax.iota(jnp.int32, 16)

        def base_of(g):
            return pl.multiple_of(wid * rows_per_w + g * CHUNK, CHUNK)

        def transpose_chunk(g, b):
            gbase = g * (CHUNK * chars)
            for j in range(chars):
                for c0 in range(0, CHUNK, 16):
                    gidx = gbase + (c0 + lanes) * chars + j
                    src = plsc.load_gather(idx_raw, [gidx])
                    idx_t[pl.ds(b * idx_rows * 128 + j * 128 + c0, 16)] = src

        def start_gathers(b):
            for j in range(idx_rows):
                pltpu.async_copy(
                    table_hbm.at[idx_t.at[pl.ds((b * idx_rows + j) * 128, 128)]],
                    acc_v.at[b],
                    sem_g[b],
                    add=True,
                )

        def wait_gathers(b):
            for _ in range(idx_rows):
                pltpu.make_async_copy(
                    table_hbm.at[idx_t.at[pl.ds(0, 128)]], acc_v.at[b],
                    sem_g[b]).wait()

        def out_copy(g, b):
            return pltpu.make_async_copy(
                acc_v.at[b], out_hbm.at[pl.ds(base_of(g), CHUNK)], sem_o[b])

        zero = jnp.zeros((16,), jnp.float32)

        def zero_acc(b):
            av = acc_v.at[b]

            def zb(c, carry):
                for d in range(dim // 16):
                    av[c, pl.ds(d * 16, 16)] = zero
                return carry

            lax.fori_loop(0, CHUNK, zb, 0, unroll=4)

        def pair_body(gg, carry):
            for b in range(2):
                g = gg * 2 + b
                nb = 1 - b

                @pl.when(g >= 2)
                def _():
                    out_copy(g - 2, b).wait()

                zero_acc(b)
                transpose_chunk(g, b)
                start_gathers(b)

                @pl.when(g >= 1)
                def _():
                    wait_gathers(nb)
                    out_copy(g - 1, nb).start()

            return carry

        lax.fori_loop(0, n_chunks // 2, pair_body, 0)
        wait_gathers(1)
        out_copy(n_chunks - 1, 1).start()
        out_copy(n_chunks - 2, 0).wait()
        out_copy(n_chunks - 1, 1).wait()

    return embed_sum


def kernel(morphemes, table):
    b, s, chars = morphemes.shape
    vocab, dim = table.shape
    n_rows = b * s
    idx1d = morphemes.reshape(n_rows * chars)
    fn = _make_sc_kernel(n_rows, chars, vocab, dim)
    out = fn(idx1d, table)
    return out.reshape(b, s, dim)
